# baseline (speedup 1.0000x reference)
import functools

import jax
import jax.numpy as jnp
from jax import lax
from jax.experimental import pallas as pl
from jax.experimental.pallas import tpu as pltpu
from jax.experimental.pallas import tpu_sc as plsc

BB, CC, NN, KK = 8, 128, 2048, 8
NKT = NN * KK
RBLK = 512
_NC, _NS = 2, 16
_NW = _NC * _NS
_CPT = CC // _NW


def _prep_body(pts_full_ref, pts_blk_ref, w2_ref, feat_ref, t_ref, idx_ref):
    pts_full = pts_full_ref[0]
    pts_blk = pts_blk_ref[0]
    feat = jnp.where(pts_blk >= 0, pts_blk, 0.01 * pts_blk)
    feat_ref[0] = feat
    t_ref[0] = lax.dot_general(w2_ref[...], feat, (((1,), (0,)), ((), ())))
    xx = jnp.sum(pts_full * pts_full, axis=0, keepdims=True)
    inner = -2.0 * lax.dot_general(
        pts_blk, pts_full, (((0,), (0,)), ((), ())))
    ones = jnp.ones((CC, 1), jnp.float32)
    xxr = lax.dot_general(
        pts_blk * pts_blk, ones, (((0,), (0,)), ((), ())))
    pw = (-xx - inner) - xxr
    colf = lax.broadcasted_iota(jnp.int32, (RBLK, NN), 1).astype(jnp.float32)
    cols = []
    for _ in range(KK):
        m = jnp.max(pw, axis=1, keepdims=True)
        cand = jnp.where(pw == m, colf, float(NN))
        af = jnp.min(cand, axis=1, keepdims=True)
        cols.append(af.astype(jnp.int32))
        pw = jnp.where(colf == af, -jnp.inf, pw)
    idx_ref[0] = jnp.concatenate(cols, axis=1)


def _make_prep(nb):
    return pl.pallas_call(
        _prep_body,
        grid=(nb, NN // RBLK),
        in_specs=[
            pl.BlockSpec((1, CC, NN), lambda b, r: (b, 0, 0)),
            pl.BlockSpec((1, CC, RBLK), lambda b, r: (b, 0, r)),
            pl.BlockSpec((CC, CC), lambda b, r: (0, 0)),
        ],
        out_specs=[
            pl.BlockSpec((1, CC, RBLK), lambda b, r: (b, 0, r)),
            pl.BlockSpec((1, CC, RBLK), lambda b, r: (b, 0, r)),
            pl.BlockSpec((1, RBLK, KK), lambda b, r: (b, r, 0)),
        ],
        out_shape=[
            jax.ShapeDtypeStruct((nb, CC, NN), jnp.float32),
            jax.ShapeDtypeStruct((nb, CC, NN), jnp.float32),
            jax.ShapeDtypeStruct((nb, NN, KK), jnp.int32),
        ],
    )


def _sc_body_rows(_ROWS, feat_hbm, t_hbm, idxf_hbm, pts_hbm,
             g_hbm, gnn_hbm, o_hbm,
             idxf_v,
             srcf_a, srct_a, pts_a, g_a, g2_a, o_a,
             srcf_b, srct_b, pts_b, g_b, g2_b, o_b,
             sem_ra, sem_rb, sem_wa, sem_wb):
    cid = lax.axis_index("c")
    sid = lax.axis_index("s")
    wid = sid * _NC + cid
    lane = lax.iota(jnp.int32, 16)

    bufs = ((srcf_a, srct_a, pts_a, g_a, g2_a, o_a, sem_ra, sem_wa),
            (srcf_b, srct_b, pts_b, g_b, g2_b, o_b, sem_rb, sem_wb))

    def rc(r):
        return r // _CPT, wid * _CPT + (r % _CPT)

    def issue_reads(r, P):
        b, c = rc(r)
        srcf, srct, pts, _, _, _, sem_r, _ = bufs[P]
        pltpu.async_copy(feat_hbm.at[b, c], srcf, sem_r)
        pltpu.async_copy(t_hbm.at[b, c], srct, sem_r)
        pltpu.async_copy(pts_hbm.at[b, c], pts, sem_r)

    def drain_reads(r, P):
        b, c = rc(r)
        srcf, srct, pts, _, _, _, sem_r, _ = bufs[P]
        pltpu.make_async_copy(feat_hbm.at[b, c], srcf, sem_r).wait()
        pltpu.make_async_copy(t_hbm.at[b, c], srct, sem_r).wait()
        pltpu.make_async_copy(pts_hbm.at[b, c], pts, sem_r).wait()

    def drain_writes(r, P):
        b, c = rc(r)
        _, _, _, g, g2, o, _, sem_w = bufs[P]
        pltpu.make_async_copy(g, g_hbm.at[b, c], sem_w).wait()
        pltpu.make_async_copy(g2, gnn_hbm.at[b, c], sem_w).wait()
        pltpu.make_async_copy(o, o_hbm.at[b, c], sem_w).wait()

    def do_row(r, P):
        b, c = rc(r)
        srcf, srct, pts, g, g2, o, _, sem_w = bufs[P]

        @pl.when(r + 1 < _ROWS)
        def _():
            issue_reads(r + 1, P ^ 1)

        @pl.when(jnp.logical_and(r % _CPT == 0, r > 0))
        def _():
            pltpu.sync_copy(idxf_hbm.at[r // _CPT], idxf_v)

        drain_reads(r, P)

        @pl.when(r >= 2)
        def _():
            drain_writes(r - 2, P)

        @plsc.parallel_loop(0, NKT, step=16, unroll=16)
        def _(s):
            iv = idxf_v[pl.ds(s, 16)]
            g[pl.ds(s, 16)] = plsc.load_gather(srcf, [iv])

        pltpu.async_copy(g, g_hbm.at[b, c], sem_w)

        @plsc.parallel_loop(0, NKT, step=16, unroll=16)
        def _(s):
            iv = idxf_v[pl.ds(s, 16)]
            g2[pl.ds(s, 16)] = plsc.load_gather(srct, [iv])

        pltpu.async_copy(g2, gnn_hbm.at[b, c], sem_w)

        @plsc.parallel_loop(0, NN, step=16, unroll=8)
        def _(s):
            sl = pl.ds(s, 16)
            base = (lane + s) * KK
            acc = srcf[sl]
            for j in range(KK):
                iv = plsc.load_gather(idxf_v, [base + j])
                acc = acc + plsc.load_gather(srcf, [iv])
            o[sl] = acc * (1.0 / 9.0) + pts[sl]

        pltpu.async_copy(o, o_hbm.at[b, c], sem_w)

    pltpu.sync_copy(idxf_hbm.at[0], idxf_v)
    issue_reads(0, 0)

    def pair(p, u):
        do_row(2 * p, 0)
        do_row(2 * p + 1, 1)
        return u

    lax.fori_loop(0, _ROWS // 2, pair, 0)
    drain_writes(_ROWS - 2, 0)
    drain_writes(_ROWS - 1, 1)


@functools.cache
def _gather_sc(nb):
    return pl.kernel(
        functools.partial(_sc_body_rows, nb * _CPT),
        out_type=(
            jax.ShapeDtypeStruct((nb, CC, NKT), jnp.float32),
            jax.ShapeDtypeStruct((nb, CC, NKT), jnp.float32),
            jax.ShapeDtypeStruct((nb, CC, NN), jnp.float32),
        ),
        mesh=plsc.VectorSubcoreMesh(core_axis_name="c", subcore_axis_name="s",
                                    num_cores=_NC, num_subcores=_NS),
        compiler_params=pltpu.CompilerParams(needs_layout_passes=False),
        scratch_types=(
            [pltpu.VMEM((NKT,), jnp.int32)]
            + [pltpu.VMEM((NN,), jnp.float32),
               pltpu.VMEM((NN,), jnp.float32),
               pltpu.VMEM((NN,), jnp.float32),
               pltpu.VMEM((NKT,), jnp.float32),
               pltpu.VMEM((NKT,), jnp.float32),
               pltpu.VMEM((NN,), jnp.float32)] * 2
            + [pltpu.SemaphoreType.DMA] * 4
        ),
    )


def kernel(points, W, W2, k):
    del W
    pts = points + jnp.asarray(k - KK, points.dtype)
    feat, t, idx = _make_prep(BB)(pts, pts, W2)
    idxf = idx.reshape(BB, NKT)
    g, gnn, out = _gather_sc(BB)(feat, t, idxf, pts)
    center = feat[:, :, :, None]
    return (out, center,
            g.reshape(BB, CC, NN, KK), gnn.reshape(BB, CC, NN, KK))


# speedup vs baseline: 1.0077x; 1.0077x over previous
import functools

import jax
import jax.numpy as jnp
from jax import lax
from jax.experimental import pallas as pl
from jax.experimental.pallas import tpu as pltpu
from jax.experimental.pallas import tpu_sc as plsc

BB, CC, NN, KK = 8, 128, 2048, 8
NKT = NN * KK
RBLK = 512
_NC, _NS = 2, 16
_NW = _NC * _NS
_CPT = CC // _NW


def _prep_body(pts_full_ref, pts_blk_ref, w2_ref, feat_ref, t_ref, idx_ref):
    pts_full = pts_full_ref[0]
    pts_blk = pts_blk_ref[0]
    feat = jnp.where(pts_blk >= 0, pts_blk, 0.01 * pts_blk)
    feat_ref[0] = feat
    t_ref[0] = lax.dot_general(w2_ref[...], feat, (((1,), (0,)), ((), ())))
    xx = jnp.sum(pts_full * pts_full, axis=0, keepdims=True)
    inner = -2.0 * lax.dot_general(
        pts_blk, pts_full, (((0,), (0,)), ((), ())))
    ones = jnp.ones((CC, 1), jnp.float32)
    xxr = lax.dot_general(
        pts_blk * pts_blk, ones, (((0,), (0,)), ((), ())))
    pw = (-xx - inner) - xxr
    colf = lax.broadcasted_iota(jnp.int32, (RBLK, NN), 1).astype(jnp.float32)
    cols = []
    for _ in range(KK):
        m = jnp.max(pw, axis=1, keepdims=True)
        cand = jnp.where(pw == m, colf, float(NN))
        af = jnp.min(cand, axis=1, keepdims=True)
        cols.append(af.astype(jnp.int32))
        pw = jnp.where(colf == af, -jnp.inf, pw)
    idx_ref[0] = jnp.concatenate(cols, axis=1)


def _make_prep(nb):
    return pl.pallas_call(
        _prep_body,
        grid=(nb, NN // RBLK),
        in_specs=[
            pl.BlockSpec((1, CC, NN), lambda b, r: (b, 0, 0)),
            pl.BlockSpec((1, CC, RBLK), lambda b, r: (b, 0, r)),
            pl.BlockSpec((CC, CC), lambda b, r: (0, 0)),
        ],
        out_specs=[
            pl.BlockSpec((1, CC, RBLK), lambda b, r: (b, 0, r)),
            pl.BlockSpec((1, CC, RBLK), lambda b, r: (b, 0, r)),
            pl.BlockSpec((1, RBLK, KK), lambda b, r: (b, r, 0)),
        ],
        out_shape=[
            jax.ShapeDtypeStruct((nb, CC, NN), jnp.float32),
            jax.ShapeDtypeStruct((nb, CC, NN), jnp.float32),
            jax.ShapeDtypeStruct((nb, NN, KK), jnp.int32),
        ],
    )


def _sc_body_rows(_ROWS, feat_hbm, t_hbm, idxf_hbm, idxt_hbm, pts_hbm,
             g_hbm, gnn_hbm, o_hbm,
             idxf_v, idxt_v,
             srcf_a, srct_a, pts_a, g_a, g2_a, o_a,
             srcf_b, srct_b, pts_b, g_b, g2_b, o_b,
             sem_ra, sem_rb, sem_wa, sem_wb):
    cid = lax.axis_index("c")
    sid = lax.axis_index("s")
    wid = sid * _NC + cid

    bufs = ((srcf_a, srct_a, pts_a, g_a, g2_a, o_a, sem_ra, sem_wa),
            (srcf_b, srct_b, pts_b, g_b, g2_b, o_b, sem_rb, sem_wb))

    def rc(r):
        return r // _CPT, wid * _CPT + (r % _CPT)

    def issue_reads(r, P):
        b, c = rc(r)
        srcf, srct, pts, _, _, _, sem_r, _ = bufs[P]
        pltpu.async_copy(feat_hbm.at[b, c], srcf, sem_r)
        pltpu.async_copy(t_hbm.at[b, c], srct, sem_r)
        pltpu.async_copy(pts_hbm.at[b, c], pts, sem_r)

    def drain_reads(r, P):
        b, c = rc(r)
        srcf, srct, pts, _, _, _, sem_r, _ = bufs[P]
        pltpu.make_async_copy(feat_hbm.at[b, c], srcf, sem_r).wait()
        pltpu.make_async_copy(t_hbm.at[b, c], srct, sem_r).wait()
        pltpu.make_async_copy(pts_hbm.at[b, c], pts, sem_r).wait()

    def drain_writes(r, P):
        b, c = rc(r)
        _, _, _, g, g2, o, _, sem_w = bufs[P]
        pltpu.make_async_copy(g, g_hbm.at[b, c], sem_w).wait()
        pltpu.make_async_copy(g2, gnn_hbm.at[b, c], sem_w).wait()
        pltpu.make_async_copy(o, o_hbm.at[b, c], sem_w).wait()

    def do_row(r, P):
        b, c = rc(r)
        srcf, srct, pts, g, g2, o, _, sem_w = bufs[P]

        @pl.when(r + 1 < _ROWS)
        def _():
            issue_reads(r + 1, P ^ 1)

        @pl.when(jnp.logical_and(r % _CPT == 0, r > 0))
        def _():
            bb = r // _CPT
            pltpu.sync_copy(idxf_hbm.at[bb], idxf_v)
            pltpu.sync_copy(idxt_hbm.at[bb], idxt_v)

        drain_reads(r, P)

        @pl.when(r >= 2)
        def _():
            drain_writes(r - 2, P)

        @plsc.parallel_loop(0, NKT, step=16, unroll=16)
        def _(s):
            iv = idxf_v[pl.ds(s, 16)]
            g[pl.ds(s, 16)] = plsc.load_gather(srcf, [iv])

        pltpu.async_copy(g, g_hbm.at[b, c], sem_w)

        @plsc.parallel_loop(0, NKT, step=16, unroll=16)
        def _(s):
            iv = idxf_v[pl.ds(s, 16)]
            g2[pl.ds(s, 16)] = plsc.load_gather(srct, [iv])

        pltpu.async_copy(g2, gnn_hbm.at[b, c], sem_w)

        @plsc.parallel_loop(0, NN, step=16, unroll=8)
        def _(s):
            sl = pl.ds(s, 16)
            acc = srcf[sl]
            for j in range(KK):
                iv = idxt_v[pl.ds(j * NN + s, 16)]
                acc = acc + plsc.load_gather(srcf, [iv])
            o[sl] = acc * (1.0 / 9.0) + pts[sl]

        pltpu.async_copy(o, o_hbm.at[b, c], sem_w)

    pltpu.sync_copy(idxf_hbm.at[0], idxf_v)
    pltpu.sync_copy(idxt_hbm.at[0], idxt_v)
    issue_reads(0, 0)

    def pair(p, u):
        do_row(2 * p, 0)
        do_row(2 * p + 1, 1)
        return u

    lax.fori_loop(0, _ROWS // 2, pair, 0)
    drain_writes(_ROWS - 2, 0)
    drain_writes(_ROWS - 1, 1)


@functools.cache
def _gather_sc(nb):
    return pl.kernel(
        functools.partial(_sc_body_rows, nb * _CPT),
        out_type=(
            jax.ShapeDtypeStruct((nb, CC, NKT), jnp.float32),
            jax.ShapeDtypeStruct((nb, CC, NKT), jnp.float32),
            jax.ShapeDtypeStruct((nb, CC, NN), jnp.float32),
        ),
        mesh=plsc.VectorSubcoreMesh(core_axis_name="c", subcore_axis_name="s",
                                    num_cores=_NC, num_subcores=_NS),
        compiler_params=pltpu.CompilerParams(needs_layout_passes=False),
        scratch_types=(
            [pltpu.VMEM((NKT,), jnp.int32)] * 2
            + [pltpu.VMEM((NN,), jnp.float32),
               pltpu.VMEM((NN,), jnp.float32),
               pltpu.VMEM((NN,), jnp.float32),
               pltpu.VMEM((NKT,), jnp.float32),
               pltpu.VMEM((NKT,), jnp.float32),
               pltpu.VMEM((NN,), jnp.float32)] * 2
            + [pltpu.SemaphoreType.DMA] * 4
        ),
    )


def kernel(points, W, W2, k):
    del W
    pts = points + jnp.asarray(k - KK, points.dtype)
    feat, t, idx = _make_prep(BB)(pts, pts, W2)
    idxf = idx.reshape(BB, NKT)
    idxt = jnp.swapaxes(idx, 1, 2).reshape(BB, NKT)
    g, gnn, out = _gather_sc(BB)(feat, t, idxf, idxt, pts)
    center = feat[:, :, :, None]
    return (out, center,
            g.reshape(BB, CC, NN, KK), gnn.reshape(BB, CC, NN, KK))
